# baseline (speedup 1.0000x reference)
import functools

import jax
import jax.numpy as jnp
from jax import lax
from jax.experimental import pallas as pl
from jax.experimental.pallas import tpu as pltpu
from jax.experimental.pallas import tpu_sc as plsc

B = 16384
D = 32
H = 128
O = 5
BB = 2048


@functools.cache
def _gather_fn():
    info = plsc.get_sparse_core_info()
    NC, NS = info.num_cores, info.num_subcores
    NW = NC * NS
    b_per_w = B // NW
    mesh = plsc.VectorSubcoreMesh(core_axis_name="c", subcore_axis_name="s")

    @functools.partial(
        pl.kernel,
        mesh=mesh,
        out_type=[
            jax.ShapeDtypeStruct((B, D), jnp.float32),
            jax.ShapeDtypeStruct((B, D), jnp.float32),
        ],
        scratch_types=[
            pltpu.VMEM((b_per_w,), jnp.int32),
            pltpu.VMEM((b_per_w, D), jnp.float32),
            pltpu.VMEM((b_per_w,), jnp.int32),
            pltpu.VMEM((b_per_w, D), jnp.float32),
            pltpu.SemaphoreType.DMA,
            pltpu.SemaphoreType.DMA,
        ],
        compiler_params=pltpu.CompilerParams(use_tc_tiling_on_sc=False),
    )
    def gather_k(utab, mtab, uid, mid, ue_out, me_out,
                 uidx_v, urows_v, midx_v, mrows_v, usem, msem):
        wid = lax.axis_index("s") * NC + lax.axis_index("c")
        base = wid * b_per_w
        pltpu.sync_copy(uid.at[pl.ds(base, b_per_w)], uidx_v)
        pltpu.sync_copy(mid.at[pl.ds(base, b_per_w)], midx_v)
        cu = pltpu.async_copy(utab.at[uidx_v], urows_v, usem)
        cm = pltpu.async_copy(mtab.at[midx_v], mrows_v, msem)
        cu.wait()
        cm.wait()
        pltpu.sync_copy(urows_v, ue_out.at[pl.ds(base, b_per_w)])
        pltpu.sync_copy(mrows_v, me_out.at[pl.ds(base, b_per_w)])

    return gather_k


def _mlp_body(ue, me, w1u, w1m, b1, w2t, b2, woutt, bout, out):
    h1 = jnp.maximum(
        jnp.dot(ue[...], w1u[...], preferred_element_type=jnp.float32)
        + jnp.dot(me[...], w1m[...], preferred_element_type=jnp.float32)
        + b1[...], 0.0)
    h2 = jnp.maximum(
        jnp.dot(h1, w2t[...], preferred_element_type=jnp.float32) + b2[...], 0.0)
    out[...] = jnp.dot(h2, woutt[...], preferred_element_type=jnp.float32) + bout[...]


def kernel(User_ID, Movie_ID, Rating, user_table, movie_table,
           W1, b1, W2, b2, Wout, bout):
    ue, me = _gather_fn()(user_table, movie_table, User_ID, Movie_ID)

    w1u = W1[:, :D].T
    w1m = W1[:, D:].T
    w2t = W2.T
    woutt = Wout.T

    out = pl.pallas_call(
        _mlp_body,
        grid=(B // BB,),
        in_specs=[
            pl.BlockSpec((BB, D), lambda i: (i, 0)),
            pl.BlockSpec((BB, D), lambda i: (i, 0)),
            pl.BlockSpec((D, H), lambda i: (0, 0)),
            pl.BlockSpec((D, H), lambda i: (0, 0)),
            pl.BlockSpec((1, H), lambda i: (0, 0)),
            pl.BlockSpec((H, H), lambda i: (0, 0)),
            pl.BlockSpec((1, H), lambda i: (0, 0)),
            pl.BlockSpec((H, O), lambda i: (0, 0)),
            pl.BlockSpec((1, O), lambda i: (0, 0)),
        ],
        out_specs=pl.BlockSpec((BB, O), lambda i: (i, 0)),
        out_shape=jax.ShapeDtypeStruct((B, O), jnp.float32),
    )(ue, me, w1u, w1m, b1.reshape(1, H), w2t, b2.reshape(1, H),
      woutt, bout.reshape(1, O))
    return out
